# baseline (speedup 1.0000x reference)
import functools

import jax
import jax.numpy as jnp
from jax import lax
from jax.experimental import pallas as pl
from jax.experimental.pallas import tpu as pltpu
from jax.experimental.pallas import tpu_sc as plsc

ROWS = 16384
COLS = 50
D = 128
B = ROWS * COLS
TROWS = 49
NC = 2
NS = 16
NW = NC * NS
BPW = B // NW
CHUNK = 256
NSTEPS = BPW // CHUNK
HALF = NSTEPS // 2

_mesh = plsc.VectorSubcoreMesh(core_axis_name="c", subcore_axis_name="s")


@functools.partial(
    pl.kernel,
    mesh=_mesh,
    out_type=jax.ShapeDtypeStruct((B, D), jnp.float32),
    scratch_types=[
        pltpu.VMEM_SHARED((TROWS, D), jnp.float32),
        pltpu.VMEM((BPW,), jnp.int32),
        pltpu.VMEM((2, CHUNK, D), jnp.float32),
        pltpu.SemaphoreType.DMA,
        pltpu.SemaphoreType.DMA,
        pltpu.SemaphoreType.DMA,
        pltpu.SemaphoreType.DMA,
    ],
)
def _emb_lookup(idx_hbm, table_hbm, out_hbm, table_sh, idx_v, rbuf, sg0, sg1, so0, so1):
    sid = lax.axis_index("s")
    wid = sid * NC + lax.axis_index("c")
    base = wid * BPW

    @pl.when(sid == 0)
    def _():
        pltpu.sync_copy(table_hbm, table_sh)

    pltpu.sync_copy(idx_hbm.at[pl.ds(base, BPW)], idx_v)
    plsc.subcore_barrier()

    sg = (sg0, sg1)
    so = (so0, so1)

    def body(j, carry):
        for b in range(2):
            off = (2 * j + b) * CHUNK

            @pl.when(j >= 1)
            def _():
                pltpu.make_async_copy(
                    rbuf.at[b],
                    out_hbm.at[pl.ds(base + off - 2 * CHUNK, CHUNK)],
                    so[b],
                ).wait()

            pltpu.async_copy(
                table_sh.at[idx_v.at[pl.ds(off, CHUNK)]], rbuf.at[b], sg[b]
            ).wait()
            pltpu.async_copy(
                rbuf.at[b], out_hbm.at[pl.ds(base + off, CHUNK)], so[b]
            )
        return carry

    lax.fori_loop(0, HALF, body, 0)
    for b in range(2):
        off = (NSTEPS - 2 + b) * CHUNK
        pltpu.make_async_copy(
            rbuf.at[b], out_hbm.at[pl.ds(base + off, CHUNK)], so[b]
        ).wait()


def kernel(time, table):
    idx = time.reshape(B).astype(jnp.int32)
    out = _emb_lookup(idx, table)
    return out.reshape(ROWS, COLS, D)
